# baseline (speedup 1.0000x reference)
import functools

import jax
import jax.numpy as jnp
from jax import lax
from jax.experimental import pallas as pl
from jax.experimental.pallas import tpu as pltpu
from jax.experimental.pallas import tpu_sc as plsc

NUM_STAGES = 4
K = 8192
D = 256
N_TOK = 4608

M_TILE = 1152
N_TILE = 2048
N_M = N_TOK // M_TILE
N_N = K // N_TILE

_NC = 2
_NS = 16
_NW = _NC * _NS
_BPW = N_TOK // _NW
_CH = 72

_INV_N = 1.0 / (N_TOK * D)


def _ulp_band(dmin):
    dp = jnp.reshape(dmin, (M_TILE // 128, 128))
    mstar = jnp.sqrt(jnp.maximum(dp, 0.0))
    bits = lax.bitcast_convert_type(dp, jnp.int32)
    pos = dp > 0.0
    bb = jnp.where(pos, dp, 0.0)
    for k in range(1, 7):
        cand = lax.bitcast_convert_type(bits + k, jnp.float32)
        ok = pos & (jnp.sqrt(cand) == mstar)
        bb = jnp.where(ok, cand, bb)
    return jnp.reshape(bb, (M_TILE, 1))


def _fused_argmin(cur2, cb_ref, cn_ref, enc_ref, d2_ref):
    rn = 0.25 * jnp.sum(cur2 * cur2, axis=1, keepdims=True)
    dmin = None
    for c in range(N_N):
        cb = cb_ref[pl.ds(c * N_TILE, N_TILE), :]
        mm2 = lax.dot_general(cur2, cb, (((1,), (1,)), ((), ())),
                              preferred_element_type=jnp.float32)
        d2 = (rn + mm2) + cn_ref[:, pl.ds(c * N_TILE, N_TILE)]
        d2_ref[:, pl.ds(c * N_TILE, N_TILE)] = d2
        rmin = jnp.min(d2, axis=1, keepdims=True)
        dmin = rmin if c == 0 else jnp.minimum(dmin, rmin)
    bb = _ulp_band(dmin)
    BIAS = 0x4B000000
    kmin = None
    for c in range(N_N):
        d2 = d2_ref[:, pl.ds(c * N_TILE, N_TILE)]
        io = lax.broadcasted_iota(jnp.int32, (M_TILE, N_TILE), 1) + (BIAS + c * N_TILE)
        key = lax.bitcast_convert_type(jnp.where(d2 <= bb, io, jnp.int32(BIAS + K)),
                                       jnp.float32)
        r = jnp.min(key, axis=1, keepdims=True)
        kmin = r if c == 0 else jnp.minimum(kmin, r)
    idx = lax.bitcast_convert_type(kmin, jnp.int32) - BIAS
    enc_ref[...] = jnp.broadcast_to(idx, (M_TILE, 128))


def _body_first(x_ref, cb_ref, cn_ref, enc_ref, d2_ref):
    _fused_argmin(-2.0 * x_ref[...], cb_ref, cn_ref, enc_ref, d2_ref)


def _body_mid(curp_ref, near_ref, quantp_ref, cb_ref, cn_ref,
              enc_ref, cur_ref, quant_ref, ls_ref, d2_ref):
    cur = curp_ref[...] - near_ref[...]
    cur_ref[...] = cur
    quant_ref[...] = quantp_ref[...] + near_ref[...]
    cur2 = -2.0 * cur
    rn = 0.25 * jnp.sum(cur2 * cur2, axis=1, keepdims=True)
    ls_ref[...] = jnp.broadcast_to(jnp.sum(rn).reshape(1, 1, 1), (1, 1, 128))
    _fused_argmin(cur2, cb_ref, cn_ref, enc_ref, d2_ref)


def _body_last(curp_ref, near_ref, quantp_ref, qout_ref, ls_ref):
    cur = curp_ref[...] - near_ref[...]
    quant = quantp_ref[...] + near_ref[...]
    qout_ref[...] = cur + (quant - cur)
    rn = jnp.sum(cur * cur, axis=1, keepdims=True)
    ls_ref[...] = jnp.broadcast_to(jnp.sum(rn).reshape(1, 1, 1), (1, 1, 128))


_TOK_SPEC = pl.BlockSpec((M_TILE, D), lambda t: (t, 0))
_LS_SPEC = pl.BlockSpec((1, 1, 128), lambda t: (t, 0, 0))
_PARALLEL = pltpu.CompilerParams(dimension_semantics=("parallel",))


def _stage_first(x2d, cb, cn_row):
    enc2d = pl.pallas_call(
        _body_first,
        grid=(N_M,),
        in_specs=[
            _TOK_SPEC,
            pl.BlockSpec((K, D), lambda t: (0, 0)),
            pl.BlockSpec((1, K), lambda t: (0, 0)),
        ],
        out_specs=pl.BlockSpec((M_TILE, 128), lambda t: (t, 0)),
        out_shape=jax.ShapeDtypeStruct((N_TOK, 128), jnp.int32),
        scratch_shapes=[pltpu.VMEM((M_TILE, K), jnp.float32)],
        compiler_params=_PARALLEL,
    )(x2d, cb, cn_row)
    return enc2d[:, 0]


def _stage_mid(curp, near, quantp, cb, cn_row):
    enc2d, cur, quant, ls = pl.pallas_call(
        _body_mid,
        grid=(N_M,),
        in_specs=[
            _TOK_SPEC,
            _TOK_SPEC,
            _TOK_SPEC,
            pl.BlockSpec((K, D), lambda t: (0, 0)),
            pl.BlockSpec((1, K), lambda t: (0, 0)),
        ],
        out_specs=[
            pl.BlockSpec((M_TILE, 128), lambda t: (t, 0)),
            _TOK_SPEC,
            _TOK_SPEC,
            _LS_SPEC,
        ],
        out_shape=[
            jax.ShapeDtypeStruct((N_TOK, 128), jnp.int32),
            jax.ShapeDtypeStruct((N_TOK, D), jnp.float32),
            jax.ShapeDtypeStruct((N_TOK, D), jnp.float32),
            jax.ShapeDtypeStruct((N_M, 1, 128), jnp.float32),
        ],
        scratch_shapes=[pltpu.VMEM((M_TILE, K), jnp.float32)],
        compiler_params=_PARALLEL,
    )(curp, near, quantp, cb, cn_row)
    return enc2d[:, 0], cur, quant, ls


def _stage_last(curp, near, quantp):
    qout, ls = pl.pallas_call(
        _body_last,
        grid=(N_M,),
        in_specs=[_TOK_SPEC, _TOK_SPEC, _TOK_SPEC],
        out_specs=[_TOK_SPEC, _LS_SPEC],
        out_shape=[
            jax.ShapeDtypeStruct((N_TOK, D), jnp.float32),
            jax.ShapeDtypeStruct((N_M, 1, 128), jnp.float32),
        ],
        compiler_params=_PARALLEL,
    )(curp, near, quantp)
    return qout, ls


def _sc_gather_body(table_hbm, idx_hbm, out_hbm, idx_a, idx_b, rows_v, sem):
    wid = lax.axis_index("s") * _NC + lax.axis_index("c")
    base = wid * _BPW
    pltpu.sync_copy(idx_hbm.at[pl.ds(base, _CH)], idx_a)
    pltpu.sync_copy(idx_hbm.at[pl.ds(base + _CH, _CH)], idx_b)
    pltpu.async_copy(table_hbm.at[idx_a], rows_v.at[pl.ds(0, _CH)], sem).wait()
    pltpu.async_copy(table_hbm.at[idx_b], rows_v.at[pl.ds(_CH, _CH)], sem).wait()
    pltpu.sync_copy(rows_v, out_hbm.at[pl.ds(base, _BPW)])


@functools.cache
def _sc_gather():
    return pl.kernel(
        _sc_gather_body,
        mesh=plsc.VectorSubcoreMesh(core_axis_name="c", subcore_axis_name="s"),
        out_type=jax.ShapeDtypeStruct((N_TOK, D), jnp.float32),
        scratch_types=[
            pltpu.VMEM((_CH,), jnp.int32),
            pltpu.VMEM((_CH,), jnp.int32),
            pltpu.VMEM((_BPW, D), jnp.float32),
            pltpu.SemaphoreType.DMA,
        ],
    )


def kernel(x, codebooks):
    b, s, d = x.shape
    x2d = x.reshape(-1, d)
    cn = jnp.sum(codebooks * codebooks, axis=2)

    enc0 = _stage_first(x2d, codebooks[0], cn[0][None, :])
    near = _sc_gather()(codebooks[0], enc0)
    encs = [enc0]
    cur, quant = x2d, jnp.zeros_like(x2d)
    ls_parts = []
    for i in range(1, NUM_STAGES):
        enc, cur, quant, ls = _stage_mid(cur, near, quant,
                                         codebooks[i], cn[i][None, :])
        near = _sc_gather()(codebooks[i], enc)
        encs.append(enc)
        ls_parts.append(ls)
    quantised, ls = _stage_last(cur, near, quant)
    ls_parts.append(ls)

    loss = jnp.zeros((), dtype=jnp.float32)
    for ls in ls_parts:
        loss = loss + jnp.sum(ls[:, 0, 0]) * _INV_N
    discrete_enc = jnp.stack(encs, axis=-1).reshape(b, s, NUM_STAGES)
    return (loss, loss, discrete_enc, quantised.reshape(b, s, d))


# speedup vs baseline: 1.0083x; 1.0083x over previous
import functools

import jax
import jax.numpy as jnp
from jax import lax
from jax.experimental import pallas as pl
from jax.experimental.pallas import tpu as pltpu
from jax.experimental.pallas import tpu_sc as plsc

NUM_STAGES = 4
K = 8192
D = 256
N_TOK = 4608

M_TILE = 768
N_TILE = 4096
N_M = N_TOK // M_TILE
N_N = K // N_TILE

_NC = 2
_NS = 16
_NW = _NC * _NS
_BPW = N_TOK // _NW
_CH = 72

_INV_N = 1.0 / (N_TOK * D)


def _ulp_band(dmin):
    dp = jnp.reshape(dmin, (M_TILE // 128, 128))
    mstar = jnp.sqrt(jnp.maximum(dp, 0.0))
    bits = lax.bitcast_convert_type(dp, jnp.int32)
    pos = dp > 0.0
    bb = jnp.where(pos, dp, 0.0)
    for k in range(1, 7):
        cand = lax.bitcast_convert_type(bits + k, jnp.float32)
        ok = pos & (jnp.sqrt(cand) == mstar)
        bb = jnp.where(ok, cand, bb)
    return jnp.reshape(bb, (M_TILE, 1))


def _fused_argmin(cur2, cb_ref, cn_ref, enc_ref, d2_ref):
    rn = 0.25 * jnp.sum(cur2 * cur2, axis=1, keepdims=True)
    dmin = None
    for c in range(N_N):
        cb = cb_ref[pl.ds(c * N_TILE, N_TILE), :]
        mm2 = lax.dot_general(cur2, cb, (((1,), (1,)), ((), ())),
                              preferred_element_type=jnp.float32)
        d2 = (rn + mm2) + cn_ref[:, pl.ds(c * N_TILE, N_TILE)]
        d2_ref[:, pl.ds(c * N_TILE, N_TILE)] = d2
        rmin = jnp.min(d2, axis=1, keepdims=True)
        dmin = rmin if c == 0 else jnp.minimum(dmin, rmin)
    bb = _ulp_band(dmin)
    BIAS = 0x4B000000
    kmin = None
    for c in range(N_N):
        d2 = d2_ref[:, pl.ds(c * N_TILE, N_TILE)]
        io = lax.broadcasted_iota(jnp.int32, (M_TILE, N_TILE), 1) + (BIAS + c * N_TILE)
        key = lax.bitcast_convert_type(jnp.where(d2 <= bb, io, jnp.int32(BIAS + K)),
                                       jnp.float32)
        r = jnp.min(key, axis=1, keepdims=True)
        kmin = r if c == 0 else jnp.minimum(kmin, r)
    idx = lax.bitcast_convert_type(kmin, jnp.int32) - BIAS
    enc_ref[...] = jnp.broadcast_to(idx, (M_TILE, 128))


def _body_first(x_ref, cb_ref, cn_ref, enc_ref, d2_ref):
    _fused_argmin(-2.0 * x_ref[...], cb_ref, cn_ref, enc_ref, d2_ref)


def _body_mid(curp_ref, near_ref, quantp_ref, cb_ref, cn_ref,
              enc_ref, cur_ref, quant_ref, ls_ref, d2_ref):
    cur = curp_ref[...] - near_ref[...]
    cur_ref[...] = cur
    quant_ref[...] = quantp_ref[...] + near_ref[...]
    cur2 = -2.0 * cur
    rn = 0.25 * jnp.sum(cur2 * cur2, axis=1, keepdims=True)
    ls_ref[...] = jnp.broadcast_to(jnp.sum(rn).reshape(1, 1, 1), (1, 1, 128))
    _fused_argmin(cur2, cb_ref, cn_ref, enc_ref, d2_ref)


def _body_last(curp_ref, near_ref, quantp_ref, qout_ref, ls_ref):
    cur = curp_ref[...] - near_ref[...]
    quant = quantp_ref[...] + near_ref[...]
    qout_ref[...] = cur + (quant - cur)
    rn = jnp.sum(cur * cur, axis=1, keepdims=True)
    ls_ref[...] = jnp.broadcast_to(jnp.sum(rn).reshape(1, 1, 1), (1, 1, 128))


_TOK_SPEC = pl.BlockSpec((M_TILE, D), lambda t: (t, 0))
_LS_SPEC = pl.BlockSpec((1, 1, 128), lambda t: (t, 0, 0))
_PARALLEL = pltpu.CompilerParams(dimension_semantics=("parallel",))


def _stage_first(x2d, cb, cn_row):
    enc2d = pl.pallas_call(
        _body_first,
        grid=(N_M,),
        in_specs=[
            _TOK_SPEC,
            pl.BlockSpec((K, D), lambda t: (0, 0)),
            pl.BlockSpec((1, K), lambda t: (0, 0)),
        ],
        out_specs=pl.BlockSpec((M_TILE, 128), lambda t: (t, 0)),
        out_shape=jax.ShapeDtypeStruct((N_TOK, 128), jnp.int32),
        scratch_shapes=[pltpu.VMEM((M_TILE, K), jnp.float32)],
        compiler_params=_PARALLEL,
    )(x2d, cb, cn_row)
    return enc2d[:, 0]


def _stage_mid(curp, near, quantp, cb, cn_row):
    enc2d, cur, quant, ls = pl.pallas_call(
        _body_mid,
        grid=(N_M,),
        in_specs=[
            _TOK_SPEC,
            _TOK_SPEC,
            _TOK_SPEC,
            pl.BlockSpec((K, D), lambda t: (0, 0)),
            pl.BlockSpec((1, K), lambda t: (0, 0)),
        ],
        out_specs=[
            pl.BlockSpec((M_TILE, 128), lambda t: (t, 0)),
            _TOK_SPEC,
            _TOK_SPEC,
            _LS_SPEC,
        ],
        out_shape=[
            jax.ShapeDtypeStruct((N_TOK, 128), jnp.int32),
            jax.ShapeDtypeStruct((N_TOK, D), jnp.float32),
            jax.ShapeDtypeStruct((N_TOK, D), jnp.float32),
            jax.ShapeDtypeStruct((N_M, 1, 128), jnp.float32),
        ],
        scratch_shapes=[pltpu.VMEM((M_TILE, K), jnp.float32)],
        compiler_params=_PARALLEL,
    )(curp, near, quantp, cb, cn_row)
    return enc2d[:, 0], cur, quant, ls


def _stage_last(curp, near, quantp):
    qout, ls = pl.pallas_call(
        _body_last,
        grid=(N_M,),
        in_specs=[_TOK_SPEC, _TOK_SPEC, _TOK_SPEC],
        out_specs=[_TOK_SPEC, _LS_SPEC],
        out_shape=[
            jax.ShapeDtypeStruct((N_TOK, D), jnp.float32),
            jax.ShapeDtypeStruct((N_M, 1, 128), jnp.float32),
        ],
        compiler_params=_PARALLEL,
    )(curp, near, quantp)
    return qout, ls


def _sc_gather_body(table_hbm, idx_hbm, out_hbm, idx_a, idx_b, rows_v, sem):
    wid = lax.axis_index("s") * _NC + lax.axis_index("c")
    base = wid * _BPW
    pltpu.sync_copy(idx_hbm.at[pl.ds(base, _CH)], idx_a)
    pltpu.sync_copy(idx_hbm.at[pl.ds(base + _CH, _CH)], idx_b)
    pltpu.async_copy(table_hbm.at[idx_a], rows_v.at[pl.ds(0, _CH)], sem).wait()
    pltpu.async_copy(table_hbm.at[idx_b], rows_v.at[pl.ds(_CH, _CH)], sem).wait()
    pltpu.sync_copy(rows_v, out_hbm.at[pl.ds(base, _BPW)])


@functools.cache
def _sc_gather():
    return pl.kernel(
        _sc_gather_body,
        mesh=plsc.VectorSubcoreMesh(core_axis_name="c", subcore_axis_name="s"),
        out_type=jax.ShapeDtypeStruct((N_TOK, D), jnp.float32),
        scratch_types=[
            pltpu.VMEM((_CH,), jnp.int32),
            pltpu.VMEM((_CH,), jnp.int32),
            pltpu.VMEM((_BPW, D), jnp.float32),
            pltpu.SemaphoreType.DMA,
        ],
    )


def kernel(x, codebooks):
    b, s, d = x.shape
    x2d = x.reshape(-1, d)
    cn = jnp.sum(codebooks * codebooks, axis=2)

    enc0 = _stage_first(x2d, codebooks[0], cn[0][None, :])
    near = _sc_gather()(codebooks[0], enc0)
    encs = [enc0]
    cur, quant = x2d, jnp.zeros_like(x2d)
    ls_parts = []
    for i in range(1, NUM_STAGES):
        enc, cur, quant, ls = _stage_mid(cur, near, quant,
                                         codebooks[i], cn[i][None, :])
        near = _sc_gather()(codebooks[i], enc)
        encs.append(enc)
        ls_parts.append(ls)
    quantised, ls = _stage_last(cur, near, quant)
    ls_parts.append(ls)

    loss = jnp.zeros((), dtype=jnp.float32)
    for ls in ls_parts:
        loss = loss + jnp.sum(ls[:, 0, 0]) * _INV_N
    discrete_enc = jnp.stack(encs, axis=-1).reshape(b, s, NUM_STAGES)
    return (loss, loss, discrete_enc, quantised.reshape(b, s, d))
